# Initial kernel scaffold; baseline (speedup 1.0000x reference)
import jax
import jax.numpy as jnp
from jax.experimental import pallas as pl


def kernel(x, edge_index, W1, b1, W2, b2):
    raise NotImplementedError("write your pallas kernel here")


# speedup vs baseline: 22.0917x; 22.0917x over previous
import functools

import jax
import jax.numpy as jnp
from jax import lax
from jax.experimental import pallas as pl
from jax.experimental.pallas import tpu as pltpu
from jax.experimental.pallas import tpu_sc as plsc

NC = 2
NS = 16
NW = NC * NS
EB = 128
DH = 16

_MESH = dict(core_axis_name="c", subcore_axis_name="s")


@functools.partial(jax.jit, static_argnames=("n_pad", "nb"))
def _sc_degree(dst_r, zeros1, ones_eb, *, n_pad, nb):
    @functools.partial(
        pl.kernel,
        out_type=jax.ShapeDtypeStruct((NC, n_pad), jnp.float32),
        mesh=plsc.VectorSubcoreMesh(**_MESH),
        compiler_params=pltpu.CompilerParams(use_tc_tiling_on_sc=False),
        scratch_types=[
            pltpu.VMEM((EB,), jnp.int32),
            pltpu.VMEM((EB,), jnp.float32),
            pltpu.VMEM_SHARED((n_pad,), jnp.float32),
        ],
    )
    def k(dst_hbm, zeros_hbm, ones_hbm, out_hbm, idx_v, ones_v, acc_sh):
        cid = lax.axis_index("c")
        sid = lax.axis_index("s")
        wid = sid * NC + cid
        rpt = n_pad // NS
        sl = pl.ds(sid * rpt, rpt)
        pltpu.sync_copy(zeros_hbm.at[sl], acc_sh.at[sl])
        pltpu.sync_copy(ones_hbm, ones_v)
        plsc.subcore_barrier()

        def body(j, carry):
            pltpu.sync_copy(dst_hbm.at[wid, j], idx_v)
            pltpu.sync_copy(ones_v, acc_sh.at[idx_v], add=True)
            return carry

        lax.fori_loop(0, nb, body, 0)
        plsc.subcore_barrier()
        pltpu.sync_copy(acc_sh.at[sl], out_hbm.at[cid, sl])

    return k(dst_r, zeros1, ones_eb)


@functools.partial(jax.jit, static_argnames=("n_pad", "nb"))
def _sc_aggregate(g, src_r, dst_r, zeros2, *, n_pad, nb):
    @functools.partial(
        pl.kernel,
        out_type=jax.ShapeDtypeStruct((NC, n_pad, DH), jnp.float32),
        mesh=plsc.VectorSubcoreMesh(**_MESH),
        compiler_params=pltpu.CompilerParams(use_tc_tiling_on_sc=False),
        scratch_types=[
            pltpu.VMEM((EB,), jnp.int32),
            pltpu.VMEM((EB,), jnp.int32),
            pltpu.VMEM((EB, DH), jnp.float32),
            pltpu.VMEM_SHARED((n_pad, DH), jnp.float32),
            pltpu.SemaphoreType.DMA,
        ],
    )
    def k(g_hbm, src_hbm, dst_hbm, zeros_hbm, out_hbm,
          sidx_v, didx_v, rows_v, acc_sh, sem):
        cid = lax.axis_index("c")
        sid = lax.axis_index("s")
        wid = sid * NC + cid
        rpt = n_pad // NS
        sl = pl.ds(sid * rpt, rpt)
        pltpu.sync_copy(zeros_hbm.at[sl], acc_sh.at[sl])
        plsc.subcore_barrier()

        def body(j, carry):
            pltpu.sync_copy(src_hbm.at[wid, j], sidx_v)
            pltpu.sync_copy(dst_hbm.at[wid, j], didx_v)
            pltpu.async_copy(g_hbm.at[sidx_v], rows_v, sem).wait()
            pltpu.sync_copy(rows_v, acc_sh.at[didx_v], add=True)
            return carry

        lax.fori_loop(0, nb, body, 0)
        plsc.subcore_barrier()
        pltpu.sync_copy(acc_sh.at[sl], out_hbm.at[cid, sl])

    return k(g, src_r, dst_r, zeros2)


def _tc1_body(degp_ref, x_ref, w1_ref, dis_ref, g1_ref):
    deg = 1.0 + jnp.sum(degp_ref[...], axis=1, keepdims=True)
    dis = lax.rsqrt(deg)
    h1 = jnp.dot(x_ref[...], w1_ref[...], preferred_element_type=jnp.float32)
    dis_ref[...] = jnp.broadcast_to(dis, dis_ref.shape)
    g1_ref[...] = h1 * dis


def _tc2_body(s1p_ref, g1_ref, dis_ref, b1_ref, gz_ref):
    s = s1p_ref[0] + s1p_ref[1] + g1_ref[...]
    z = jnp.maximum(dis_ref[...] * s + b1_ref[...], 0.0)
    gz_ref[...] = z * dis_ref[...]


def _tc3_body(s2p_ref, gz_ref, dis_ref, w2_ref, b2_ref, out_ref):
    a2 = dis_ref[...] * (s2p_ref[0] + s2p_ref[1] + gz_ref[...])
    o = jnp.dot(a2, w2_ref[...], preferred_element_type=jnp.float32)
    out_ref[...] = jnp.maximum(o + b2_ref[...], 0.0)


def kernel(x, edge_index, W1, b1, W2, b2):
    n, d_in = x.shape
    d_hid = W1.shape[1]
    d_out = W2.shape[1]
    e = edge_index.shape[1]

    n_pad = ((n + NW * 8) + NW * 8 - 1) // (NW * 8) * (NW * 8)
    e_pad = (e + NW * EB - 1) // (NW * EB) * (NW * EB)
    nb = e_pad // (NW * EB)
    dummy = jnp.int32(n)

    src = jnp.concatenate(
        [edge_index[0], jnp.full((e_pad - e,), dummy, jnp.int32)]).reshape(NW, nb, EB)
    dst = jnp.concatenate(
        [edge_index[1], jnp.full((e_pad - e,), dummy, jnp.int32)]).reshape(NW, nb, EB)
    x_pad = jnp.pad(x, ((0, n_pad - n), (0, 0)))
    zeros1 = jnp.zeros((n_pad,), jnp.float32)
    zeros2 = jnp.zeros((n_pad, d_hid), jnp.float32)
    ones_eb = jnp.ones((EB,), jnp.float32)
    b1r = b1.reshape(1, d_hid)
    b2r = b2.reshape(1, d_out)

    degp = _sc_degree(dst, zeros1, ones_eb, n_pad=n_pad, nb=nb)
    degp_t = jnp.swapaxes(degp, 0, 1)

    dis, g1 = pl.pallas_call(
        _tc1_body,
        out_shape=(
            jax.ShapeDtypeStruct((n_pad, d_hid), jnp.float32),
            jax.ShapeDtypeStruct((n_pad, d_hid), jnp.float32),
        ),
    )(degp_t, x_pad, W1)

    s1p = _sc_aggregate(g1, src, dst, zeros2, n_pad=n_pad, nb=nb)

    gz = pl.pallas_call(
        _tc2_body,
        out_shape=jax.ShapeDtypeStruct((n_pad, d_hid), jnp.float32),
    )(s1p, g1, dis, b1r)

    s2p = _sc_aggregate(gz, src, dst, zeros2, n_pad=n_pad, nb=nb)

    out = pl.pallas_call(
        _tc3_body,
        out_shape=jax.ShapeDtypeStruct((n_pad, d_out), jnp.float32),
    )(s2p, gz, dis, W2, b2r)

    return out[:n]
